# baseline (speedup 1.0000x reference)
import functools

import jax
import jax.numpy as jnp
from jax import lax
from jax.experimental import pallas as pl
from jax.experimental.pallas import tpu as pltpu
from jax.experimental.pallas import tpu_sc as plsc

D_MODEL = 1024
NUM_CORES = 2
NUM_SUBCORES = 16
NUM_WORKERS = NUM_CORES * NUM_SUBCORES
CHUNK = 32


NBUF = 3


def _emb_body(n_chunks, per_worker, ids_hbm, table_hbm, out_hbm,
              idx_v, buf0, buf1, buf2, gsem0, gsem1, gsem2,
              psem0, psem1, psem2):
    wid = lax.axis_index("s") * NUM_CORES + lax.axis_index("c")
    base = wid * per_worker
    pltpu.sync_copy(ids_hbm.at[wid], idx_v)

    bufs = (buf0, buf1, buf2)
    gsems = (gsem0, gsem1, gsem2)
    psems = (psem0, psem1, psem2)
    gat = [None] * NBUF
    put = [None] * NBUF
    for j in range(n_chunks):
        b = j % NBUF
        if j >= NBUF:
            put[b].wait()
        gat[b] = pltpu.async_copy(table_hbm.at[idx_v.at[j]], bufs[b], gsems[b])
        if j >= 1:
            pb = (j - 1) % NBUF
            gat[pb].wait()
            put[pb] = pltpu.async_copy(
                bufs[pb], out_hbm.at[pl.ds(base + (j - 1) * CHUNK, CHUNK)],
                psems[pb])
    lb = (n_chunks - 1) % NBUF
    gat[lb].wait()
    put[lb] = pltpu.async_copy(
        bufs[lb], out_hbm.at[pl.ds(base + (n_chunks - 1) * CHUNK, CHUNK)],
        psems[lb])
    for b in range(NBUF):
        put[b].wait()


def kernel(input_ids, embedding_table):
    batch, seq = input_ids.shape
    n = batch * seq
    assert n % (NUM_WORKERS * CHUNK) == 0
    per_worker = n // NUM_WORKERS
    n_chunks = per_worker // CHUNK

    ids = input_ids.reshape(NUM_WORKERS, n_chunks, CHUNK)

    mesh = plsc.VectorSubcoreMesh(core_axis_name="c", subcore_axis_name="s")
    emb = pl.kernel(
        functools.partial(_emb_body, n_chunks, per_worker),
        mesh=mesh,
        out_type=jax.ShapeDtypeStruct((n, D_MODEL), jnp.float32),
        scratch_types=[
            pltpu.VMEM((n_chunks, CHUNK), jnp.int32),
            pltpu.VMEM((CHUNK, D_MODEL), jnp.float32),
            pltpu.VMEM((CHUNK, D_MODEL), jnp.float32),
            pltpu.VMEM((CHUNK, D_MODEL), jnp.float32),
            pltpu.SemaphoreType.DMA,
            pltpu.SemaphoreType.DMA,
            pltpu.SemaphoreType.DMA,
            pltpu.SemaphoreType.DMA,
            pltpu.SemaphoreType.DMA,
            pltpu.SemaphoreType.DMA,
        ],
    )
    out = emb(ids, embedding_table)
    return out.reshape(batch, seq, D_MODEL)
